# Initial kernel scaffold; baseline (speedup 1.0000x reference)
import jax
import jax.numpy as jnp
from jax.experimental import pallas as pl


def kernel(x, score):
    raise NotImplementedError("write your pallas kernel here")


# speedup vs baseline: 87.5629x; 87.5629x over previous
import functools

import jax
import jax.numpy as jnp
from jax import lax
from jax.experimental import pallas as pl
from jax.experimental.pallas import tpu as pltpu
from jax.experimental.pallas import tpu_sc as plsc

N_ROWS = 8192
N_COLS = 4096
N = N_ROWS * N_COLS
BLK = 8
KEEP = 4

NC = 2
NS = 16
NW = NC * NS
PER_W = N // NW
CHUNK = 16384
N_CHUNKS = PER_W // CHUNK
PANEL = 128
N_PANELS = CHUNK // PANEL


def _sc_body(x_hbm, s_hbm, o_hbm, xb, sb, ob):
    wid = lax.axis_index("s") * NC + lax.axis_index("c")
    base = wid * PER_W

    lane8 = lax.iota(jnp.int32, 16) * 8

    def do_chunk(c, _):
        off = base + c * CHUNK
        pltpu.sync_copy(x_hbm.at[pl.ds(off, CHUNK)], xb)
        pltpu.sync_copy(s_hbm.at[pl.ds(off, CHUNK)], sb)

        def do_panel(p, _):
            pbase = p * PANEL
            idx = [lane8 + (pbase + k) for k in range(BLK)]
            s = [plsc.load_gather(sb, [idx[k]]) for k in range(BLK)]
            cnt = [jnp.full((16,), k, jnp.int32) for k in range(BLK)]
            for i in range(BLK):
                for j in range(i + 1, BLK):
                    c01 = (s[j] < s[i]).astype(jnp.int32)
                    cnt[i] = cnt[i] + c01
                    cnt[j] = cnt[j] - c01
            for k in range(BLK):
                xk = plsc.load_gather(xb, [idx[k]])
                ok = jnp.where(cnt[k] >= KEEP, xk, 0.0)
                plsc.store_scatter(ob, [idx[k]], ok)
            return ()

        lax.fori_loop(0, N_PANELS, do_panel, ())
        pltpu.sync_copy(ob, o_hbm.at[pl.ds(off, CHUNK)])
        return ()

    lax.fori_loop(0, N_CHUNKS, do_chunk, ())


@jax.jit
def _sparsify(xf, sf):
    mesh = plsc.VectorSubcoreMesh(core_axis_name="c", subcore_axis_name="s")
    run = pl.kernel(
        _sc_body,
        mesh=mesh,
        out_type=jax.ShapeDtypeStruct((N,), jnp.float32),
        scratch_types=[
            pltpu.VMEM((CHUNK,), jnp.float32),
            pltpu.VMEM((CHUNK,), jnp.float32),
            pltpu.VMEM((CHUNK,), jnp.float32),
        ],
        compiler_params=pltpu.CompilerParams(needs_layout_passes=False),
    )
    return run(xf, sf)


def kernel(x, score):
    out = _sparsify(x.reshape(-1), score.reshape(-1))
    return out.reshape(N_ROWS, N_COLS)
